# baseline (speedup 1.0000x reference)
import functools

import jax
import jax.numpy as jnp
from jax import lax
from jax.experimental import pallas as pl
from jax.experimental.pallas import tpu as pltpu
from jax.experimental.pallas import tpu_sc as plsc

N = 10000
E = 320000
H = 128
G = 64
L = 3

NC = 2
NS = 16
NW = NC * NS
EDGES_PER_TILE = E // NW
CHUNK = 80
NCHUNKS = EDGES_PER_TILE // CHUNK
NBUF = 2
ROWS_PER_SUB = 632
N_PAD = NS * ROWS_PER_SUB
ZITER = ROWS_PER_SUB // CHUNK
ZREM = ROWS_PER_SUB - ZITER * CHUNK

BLK = 1000
NBLK = N // BLK


def _sc_agg(h, src, dst):
    mesh = plsc.VectorSubcoreMesh(core_axis_name="c", subcore_axis_name="s")

    @functools.partial(
        pl.kernel,
        mesh=mesh,
        out_type=jax.ShapeDtypeStruct((NC, N_PAD, H), jnp.float32),
        scratch_types=[
            pltpu.VMEM((EDGES_PER_TILE,), jnp.int32),
            pltpu.VMEM((EDGES_PER_TILE,), jnp.int32),
            pltpu.VMEM((CHUNK, H), jnp.float32),
            pltpu.VMEM((CHUNK, H), jnp.float32),
            pltpu.VMEM_SHARED((N_PAD, H), jnp.float32),
            pltpu.SemaphoreType.DMA,
            pltpu.SemaphoreType.DMA,
            pltpu.SemaphoreType.DMA,
        ],
    )
    def k(h_hbm, src_hbm, dst_hbm, out_hbm, src_all, dst_all, r0, r1,
          acc, s0, s1, sem_idx):
        cid = lax.axis_index("c")
        sid = lax.axis_index("s")
        wid = sid * NC + cid
        row0 = sid * ROWS_PER_SUB
        pltpu.async_copy(src_hbm.at[wid], src_all, sem_idx)
        pltpu.async_copy(dst_hbm.at[wid], dst_all, sem_idx)

        @pl.loop(0, CHUNK)
        def _(r):
            @pl.loop(0, H // 16)
            def _(j):
                r0[r, pl.ds(j * 16, 16)] = jnp.zeros((16,), jnp.float32)

        @pl.loop(0, ZITER)
        def _(j):
            pltpu.sync_copy(r0, acc.at[pl.ds(row0 + j * CHUNK, CHUNK)])

        pltpu.sync_copy(r0.at[pl.ds(0, ZREM)],
                        acc.at[pl.ds(row0 + ZITER * CHUNK, ZREM)])

        pltpu.make_async_copy(src_hbm.at[wid], src_all, sem_idx).wait()
        pltpu.make_async_copy(dst_hbm.at[wid], dst_all, sem_idx).wait()

        bufs = ((r0, s0), (r1, s1))

        def gstart(c, b):
            pltpu.async_copy(h_hbm.at[src_all.at[pl.ds(c * CHUNK, CHUNK)]],
                             bufs[b][0], bufs[b][1])

        def gwait(c, b):
            pltpu.make_async_copy(
                h_hbm.at[src_all.at[pl.ds(c * CHUNK, CHUNK)]],
                bufs[b][0], bufs[b][1]).wait()

        def scat(c, b):
            pltpu.sync_copy(bufs[b][0],
                            acc.at[dst_all.at[pl.ds(c * CHUNK, CHUNK)]],
                            add=True)

        for b in range(NBUF):
            gstart(b, b)
        plsc.subcore_barrier()

        NLOOP = NCHUNKS // NBUF

        @pl.loop(0, NLOOP)
        def _(i):
            c0 = i * NBUF
            for b in range(NBUF):
                cc = c0 + b
                gwait(cc, b)
                scat(cc, b)
                nxt = cc + NBUF

                @pl.when(nxt < NCHUNKS)
                def _(nxt=nxt, b=b):
                    gstart(nxt, b)

        for b in range(NCHUNKS - NLOOP * NBUF):
            cc = NLOOP * NBUF + b
            gwait(cc, b)
            scat(cc, b)

        plsc.subcore_barrier()
        pltpu.sync_copy(acc.at[pl.ds(row0, ROWS_PER_SUB)],
                        out_hbm.at[cid, pl.ds(row0, ROWS_PER_SUB)])

    return k(h, src, dst)


def _stats_to_scale_shift(s_ref, g_ref, b_ref):
    mu = s_ref[0, :] * (1.0 / N)
    var = s_ref[1, :] * (1.0 / N) - mu * mu
    inv = lax.rsqrt(var + 1e-5)
    scale = inv * g_ref[0, :]
    shift = b_ref[0, :] - mu * scale
    return scale, shift


def _mlp(x, w1_ref, b1_ref, w2_ref, b2_ref):
    t = jnp.dot(x.astype(jnp.bfloat16),
                w1_ref[...].astype(jnp.bfloat16),
                preferred_element_type=jnp.float32)
    t = jnp.maximum(t + b1_ref[...], 0.0)
    t = jnp.dot(t.astype(jnp.bfloat16),
                w2_ref[...].astype(jnp.bfloat16),
                preferred_element_type=jnp.float32)
    t = t + b2_ref[...]
    return jnp.where(t > 0.0, t, jnp.exp(jnp.minimum(t, 0.0)) - 1.0)


def _emit_stats(i, m, s_ref):
    srow = jnp.sum(m, axis=0, keepdims=True)
    sqrow = jnp.sum(m * m, axis=0, keepdims=True)
    stats = jnp.concatenate([srow, sqrow, jnp.zeros((6, H), jnp.float32)],
                            axis=0)

    @pl.when(i == 0)
    def _():
        s_ref[...] = stats

    @pl.when(i != 0)
    def _():
        s_ref[...] += stats


def _tc_mlp0(x, agg, W1, b1, W2, b2):
    def body(x_ref, agg_ref, w1_ref, b1_ref, w2_ref, b2_ref, m_ref, s_ref):
        i = pl.program_id(0)
        xx = x_ref[...] + agg_ref[0] + agg_ref[1]
        m = _mlp(xx, w1_ref, b1_ref, w2_ref, b2_ref)
        m_ref[...] = m
        _emit_stats(i, m, s_ref)

    return pl.pallas_call(
        body,
        grid=(NBLK,),
        in_specs=[
            pl.BlockSpec((BLK, H), lambda i: (i, 0)),
            pl.BlockSpec((NC, BLK, H), lambda i: (0, i, 0)),
            pl.BlockSpec((H, H), lambda i: (0, 0)),
            pl.BlockSpec((1, H), lambda i: (0, 0)),
            pl.BlockSpec((H, H), lambda i: (0, 0)),
            pl.BlockSpec((1, H), lambda i: (0, 0)),
        ],
        out_specs=[
            pl.BlockSpec((BLK, H), lambda i: (i, 0)),
            pl.BlockSpec((8, H), lambda i: (0, 0)),
        ],
        out_shape=[
            jax.ShapeDtypeStruct((N, H), jnp.float32),
            jax.ShapeDtypeStruct((8, H), jnp.float32),
        ],
    )(x, agg, W1, b1.reshape(1, H), W2, b2.reshape(1, H))


def _tc_mlpN(m_prev, aggm, stats_prev, indeg, g_prev, b_prev,
             W1, b1, W2, b2):
    def body(m_ref, agg_ref, s_ref, d_ref, g_ref, bb_ref,
             w1_ref, b1_ref, w2_ref, b2_ref, m_out, s_out):
        i = pl.program_id(0)
        scale, shift = _stats_to_scale_shift(s_ref, g_ref, bb_ref)
        deg = d_ref[0, :, 0] + d_ref[1, :, 0]
        xx = (scale[None, :] * (m_ref[...] + agg_ref[0] + agg_ref[1])
              + (1.0 + deg)[:, None] * shift[None, :])
        m = _mlp(xx, w1_ref, b1_ref, w2_ref, b2_ref)
        m_out[...] = m
        _emit_stats(i, m, s_out)

    return pl.pallas_call(
        body,
        grid=(NBLK,),
        in_specs=[
            pl.BlockSpec((BLK, H), lambda i: (i, 0)),
            pl.BlockSpec((NC, BLK, H), lambda i: (0, i, 0)),
            pl.BlockSpec((8, H), lambda i: (0, 0)),
            pl.BlockSpec((NC, BLK, H), lambda i: (0, i, 0)),
            pl.BlockSpec((1, H), lambda i: (0, 0)),
            pl.BlockSpec((1, H), lambda i: (0, 0)),
            pl.BlockSpec((H, H), lambda i: (0, 0)),
            pl.BlockSpec((1, H), lambda i: (0, 0)),
            pl.BlockSpec((H, H), lambda i: (0, 0)),
            pl.BlockSpec((1, H), lambda i: (0, 0)),
        ],
        out_specs=[
            pl.BlockSpec((BLK, H), lambda i: (i, 0)),
            pl.BlockSpec((8, H), lambda i: (0, 0)),
        ],
        out_shape=[
            jax.ShapeDtypeStruct((N, H), jnp.float32),
            jax.ShapeDtypeStruct((8, H), jnp.float32),
        ],
    )(m_prev, aggm, stats_prev, indeg, g_prev.reshape(1, H),
      b_prev.reshape(1, H), W1, b1.reshape(1, H), W2, b2.reshape(1, H))


def _tc_norm_pool(m, stats, gamma, beta, batch3, xs_in, li):
    def body(m_ref, s_ref, g_ref, bb_ref, batch_ref, xs_ref, p_ref):
        i = pl.program_id(0)
        scale, shift = _stats_to_scale_shift(s_ref, g_ref, bb_ref)
        hh = m_ref[...] * scale[None, :] + shift[None, :]
        xs_ref[...] = hh
        bt = batch_ref[0, 0, :]
        onehot = (bt[:, None] == lax.broadcasted_iota(jnp.int32, (BLK, G), 1)
                  ).astype(jnp.float32)
        pool = lax.dot_general(onehot, hh, (((0,), (0,)), ((), ())),
                               preferred_element_type=jnp.float32)

        @pl.when(i == 0)
        def _():
            p_ref[...] = pool

        @pl.when(i != 0)
        def _():
            p_ref[...] += pool

    in_specs = [
        pl.BlockSpec((BLK, H), lambda i: (i, 0)),
        pl.BlockSpec((8, H), lambda i: (0, 0)),
        pl.BlockSpec((1, H), lambda i: (0, 0)),
        pl.BlockSpec((1, H), lambda i: (0, 0)),
        pl.BlockSpec((1, 1, BLK), lambda i: (i, 0, 0)),
    ]
    inputs = [m, stats, gamma.reshape(1, H), beta.reshape(1, H), batch3]
    aliases = {}
    if li > 0:
        in_specs.append(pl.BlockSpec((BLK, H), lambda i: (0, 0)))
        inputs.append(xs_in)
        aliases = {5: 0}

    return pl.pallas_call(
        body if li == 0 else (lambda m_ref, s_ref, g_ref, bb_ref, batch_ref,
                              xs_alias, xs_ref, p_ref:
                              body(m_ref, s_ref, g_ref, bb_ref, batch_ref,
                                   xs_ref, p_ref)),
        grid=(NBLK,),
        in_specs=in_specs,
        out_specs=[
            pl.BlockSpec((BLK, H), lambda i, li=li: (i, li)),
            pl.BlockSpec((G, H), lambda i: (0, 0)),
        ],
        out_shape=[
            jax.ShapeDtypeStruct((N, L * H), jnp.float32),
            jax.ShapeDtypeStruct((G, H), jnp.float32),
        ],
        input_output_aliases=aliases,
    )(*inputs)


def kernel(x, edge_index, batch,
           W1_0, b1_0, W2_0, b2_0, gamma_0, beta_0,
           W1_1, b1_1, W2_1, b2_1, gamma_1, beta_1,
           W1_2, b1_2, W2_2, b2_2, gamma_2, beta_2):
    src = edge_index[0].reshape(NW, EDGES_PER_TILE)
    dst = edge_index[1].reshape(NW, EDGES_PER_TILE)
    batch3 = batch.reshape(NBLK, 1, BLK)

    indeg = _sc_agg(jnp.ones((N, H), jnp.float32), src, dst)

    agg0 = _sc_agg(x, src, dst)
    m0, s0 = _tc_mlp0(x, agg0, W1_0, b1_0, W2_0, b2_0)

    agg1 = _sc_agg(m0, src, dst)
    xs, pool0 = _tc_norm_pool(m0, s0, gamma_0, beta_0, batch3, None, 0)
    m1, s1 = _tc_mlpN(m0, agg1, s0, indeg, gamma_0, beta_0,
                      W1_1, b1_1, W2_1, b2_1)

    agg2 = _sc_agg(m1, src, dst)
    xs, pool1 = _tc_norm_pool(m1, s1, gamma_1, beta_1, batch3, xs, 1)
    m2, s2 = _tc_mlpN(m1, agg2, s1, indeg, gamma_1, beta_1,
                      W1_2, b1_2, W2_2, b2_2)

    xs, pool2 = _tc_norm_pool(m2, s2, gamma_2, beta_2, batch3, xs, 2)

    return (jnp.concatenate([pool0, pool1, pool2], axis=1), xs)


# speedup vs baseline: 1.0013x; 1.0013x over previous
import functools

import jax
import jax.numpy as jnp
from jax import lax
from jax.experimental import pallas as pl
from jax.experimental.pallas import tpu as pltpu
from jax.experimental.pallas import tpu_sc as plsc

N = 10000
E = 320000
H = 128
G = 64
L = 3

NC = 2
NS = 16
NW = NC * NS
EDGES_PER_TILE = E // NW
CHUNK = 80
NCHUNKS = EDGES_PER_TILE // CHUNK
NBUF = 2
ROWS_PER_SUB = 632
N_PAD = NS * ROWS_PER_SUB
ZITER = ROWS_PER_SUB // CHUNK
ZREM = ROWS_PER_SUB - ZITER * CHUNK

BLK = 1000
NBLK = N // BLK


def _sc_agg(h, src, dst):
    mesh = plsc.VectorSubcoreMesh(core_axis_name="c", subcore_axis_name="s")

    @functools.partial(
        pl.kernel,
        mesh=mesh,
        out_type=jax.ShapeDtypeStruct((NC, N_PAD, H), jnp.float32),
        scratch_types=[
            pltpu.VMEM((EDGES_PER_TILE,), jnp.int32),
            pltpu.VMEM((EDGES_PER_TILE,), jnp.int32),
            pltpu.VMEM((CHUNK, H), jnp.float32),
            pltpu.VMEM((CHUNK, H), jnp.float32),
            pltpu.VMEM_SHARED((N_PAD, H), jnp.float32),
            pltpu.SemaphoreType.DMA,
            pltpu.SemaphoreType.DMA,
            pltpu.SemaphoreType.DMA,
        ],
    )
    def k(h_hbm, src_hbm, dst_hbm, out_hbm, src_all, dst_all, r0, r1,
          acc, s0, s1, sem_idx):
        cid = lax.axis_index("c")
        sid = lax.axis_index("s")
        wid = sid * NC + cid
        row0 = sid * ROWS_PER_SUB
        pltpu.async_copy(src_hbm.at[wid], src_all, sem_idx)
        pltpu.async_copy(dst_hbm.at[wid], dst_all, sem_idx)

        @pl.loop(0, CHUNK)
        def _(r):
            @pl.loop(0, H // 16)
            def _(j):
                r0[r, pl.ds(j * 16, 16)] = jnp.zeros((16,), jnp.float32)

        @pl.loop(0, ZITER)
        def _(j):
            pltpu.sync_copy(r0, acc.at[pl.ds(row0 + j * CHUNK, CHUNK)])

        pltpu.sync_copy(r0.at[pl.ds(0, ZREM)],
                        acc.at[pl.ds(row0 + ZITER * CHUNK, ZREM)])

        pltpu.make_async_copy(src_hbm.at[wid], src_all, sem_idx).wait()
        pltpu.make_async_copy(dst_hbm.at[wid], dst_all, sem_idx).wait()

        bufs = ((r0, s0), (r1, s1))

        def gstart(c, b):
            pltpu.async_copy(h_hbm.at[src_all.at[pl.ds(c * CHUNK, CHUNK)]],
                             bufs[b][0], bufs[b][1])

        def gwait(c, b):
            pltpu.make_async_copy(
                h_hbm.at[src_all.at[pl.ds(c * CHUNK, CHUNK)]],
                bufs[b][0], bufs[b][1]).wait()

        def scat(c, b):
            pltpu.sync_copy(bufs[b][0],
                            acc.at[dst_all.at[pl.ds(c * CHUNK, CHUNK)]],
                            add=True)

        for b in range(NBUF):
            gstart(b, b)
        plsc.subcore_barrier()

        NLOOP = NCHUNKS // NBUF

        @pl.loop(0, NLOOP)
        def _(i):
            c0 = i * NBUF
            for b in range(NBUF):
                cc = c0 + b
                gwait(cc, b)
                scat(cc, b)
                nxt = cc + NBUF

                @pl.when(nxt < NCHUNKS)
                def _(nxt=nxt, b=b):
                    gstart(nxt, b)

        for b in range(NCHUNKS - NLOOP * NBUF):
            cc = NLOOP * NBUF + b
            gwait(cc, b)
            scat(cc, b)

        plsc.subcore_barrier()
        pltpu.sync_copy(acc.at[pl.ds(row0, ROWS_PER_SUB)],
                        out_hbm.at[cid, pl.ds(row0, ROWS_PER_SUB)])

    return k(h, src, dst)


def _stats_to_scale_shift(s_ref, g_ref, b_ref):
    mu = s_ref[0, :] * (1.0 / N)
    var = s_ref[1, :] * (1.0 / N) - mu * mu
    inv = lax.rsqrt(var + 1e-5)
    scale = inv * g_ref[0, :]
    shift = b_ref[0, :] - mu * scale
    return scale, shift


def _mlp(x, w1_ref, b1_ref, w2_ref, b2_ref):
    t = jnp.dot(x.astype(jnp.bfloat16),
                w1_ref[...].astype(jnp.bfloat16),
                preferred_element_type=jnp.float32)
    t = jnp.maximum(t + b1_ref[...], 0.0)
    t = jnp.dot(t.astype(jnp.bfloat16),
                w2_ref[...].astype(jnp.bfloat16),
                preferred_element_type=jnp.float32)
    t = t + b2_ref[...]
    return jnp.where(t > 0.0, t, jnp.exp(jnp.minimum(t, 0.0)) - 1.0)


def _emit_stats(i, m, s_ref):
    srow = jnp.sum(m, axis=0, keepdims=True)
    sqrow = jnp.sum(m * m, axis=0, keepdims=True)
    stats = jnp.concatenate([srow, sqrow, jnp.zeros((6, H), jnp.float32)],
                            axis=0)

    @pl.when(i == 0)
    def _():
        s_ref[...] = stats

    @pl.when(i != 0)
    def _():
        s_ref[...] += stats


def _tc_mlp0(x, agg, W1, b1, W2, b2):
    def body(x_ref, agg_ref, w1_ref, b1_ref, w2_ref, b2_ref, m_ref, s_ref):
        i = pl.program_id(0)
        xx = x_ref[...] + agg_ref[0] + agg_ref[1]
        m = _mlp(xx, w1_ref, b1_ref, w2_ref, b2_ref)
        m_ref[...] = m
        _emit_stats(i, m, s_ref)

    return pl.pallas_call(
        body,
        grid=(NBLK,),
        in_specs=[
            pl.BlockSpec((BLK, H), lambda i: (i, 0)),
            pl.BlockSpec((NC, BLK, H), lambda i: (0, i, 0)),
            pl.BlockSpec((H, H), lambda i: (0, 0)),
            pl.BlockSpec((1, H), lambda i: (0, 0)),
            pl.BlockSpec((H, H), lambda i: (0, 0)),
            pl.BlockSpec((1, H), lambda i: (0, 0)),
        ],
        out_specs=[
            pl.BlockSpec((BLK, H), lambda i: (i, 0)),
            pl.BlockSpec((8, H), lambda i: (0, 0)),
        ],
        out_shape=[
            jax.ShapeDtypeStruct((N, H), jnp.float32),
            jax.ShapeDtypeStruct((8, H), jnp.float32),
        ],
    )(x, agg, W1, b1.reshape(1, H), W2, b2.reshape(1, H))


def _tc_mlpN(m_prev, aggm, stats_prev, indeg, g_prev, b_prev,
             W1, b1, W2, b2):
    def body(m_ref, agg_ref, s_ref, d_ref, g_ref, bb_ref,
             w1_ref, b1_ref, w2_ref, b2_ref, m_out, s_out):
        i = pl.program_id(0)
        scale, shift = _stats_to_scale_shift(s_ref, g_ref, bb_ref)
        deg = d_ref[0, :, 0] + d_ref[1, :, 0]
        xx = (scale[None, :] * (m_ref[...] + agg_ref[0] + agg_ref[1])
              + (1.0 + deg)[:, None] * shift[None, :])
        m = _mlp(xx, w1_ref, b1_ref, w2_ref, b2_ref)
        m_out[...] = m
        _emit_stats(i, m, s_out)

    return pl.pallas_call(
        body,
        grid=(NBLK,),
        in_specs=[
            pl.BlockSpec((BLK, H), lambda i: (i, 0)),
            pl.BlockSpec((NC, BLK, H), lambda i: (0, i, 0)),
            pl.BlockSpec((8, H), lambda i: (0, 0)),
            pl.BlockSpec((NC, BLK, H), lambda i: (0, i, 0)),
            pl.BlockSpec((1, H), lambda i: (0, 0)),
            pl.BlockSpec((1, H), lambda i: (0, 0)),
            pl.BlockSpec((H, H), lambda i: (0, 0)),
            pl.BlockSpec((1, H), lambda i: (0, 0)),
            pl.BlockSpec((H, H), lambda i: (0, 0)),
            pl.BlockSpec((1, H), lambda i: (0, 0)),
        ],
        out_specs=[
            pl.BlockSpec((BLK, H), lambda i: (i, 0)),
            pl.BlockSpec((8, H), lambda i: (0, 0)),
        ],
        out_shape=[
            jax.ShapeDtypeStruct((N, H), jnp.float32),
            jax.ShapeDtypeStruct((8, H), jnp.float32),
        ],
    )(m_prev, aggm, stats_prev, indeg, g_prev.reshape(1, H),
      b_prev.reshape(1, H), W1, b1.reshape(1, H), W2, b2.reshape(1, H))


def _tc_norm_pool(m, stats, gamma, beta, batch3, xs_in, li):
    def body(m_ref, s_ref, g_ref, bb_ref, batch_ref, xs_ref, p_ref):
        i = pl.program_id(0)
        scale, shift = _stats_to_scale_shift(s_ref, g_ref, bb_ref)
        hh = m_ref[...] * scale[None, :] + shift[None, :]
        xs_ref[...] = hh
        bt = batch_ref[0, 0, :]
        onehot = (bt[:, None] == lax.broadcasted_iota(jnp.int32, (BLK, G), 1)
                  ).astype(jnp.float32)
        pool = lax.dot_general(onehot, hh, (((0,), (0,)), ((), ())),
                               preferred_element_type=jnp.float32)

        @pl.when(i == 0)
        def _():
            p_ref[...] = pool

        @pl.when(i != 0)
        def _():
            p_ref[...] += pool

    in_specs = [
        pl.BlockSpec((BLK, H), lambda i: (i, 0)),
        pl.BlockSpec((8, H), lambda i: (0, 0)),
        pl.BlockSpec((1, H), lambda i: (0, 0)),
        pl.BlockSpec((1, H), lambda i: (0, 0)),
        pl.BlockSpec((1, 1, BLK), lambda i: (i, 0, 0)),
    ]
    inputs = [m, stats, gamma.reshape(1, H), beta.reshape(1, H), batch3]
    aliases = {}
    if li > 0:
        in_specs.append(pl.BlockSpec((BLK, H), lambda i: (0, 0)))
        inputs.append(xs_in)
        aliases = {5: 0}

    return pl.pallas_call(
        body if li == 0 else (lambda m_ref, s_ref, g_ref, bb_ref, batch_ref,
                              xs_alias, xs_ref, p_ref:
                              body(m_ref, s_ref, g_ref, bb_ref, batch_ref,
                                   xs_ref, p_ref)),
        grid=(NBLK,),
        in_specs=in_specs,
        out_specs=[
            pl.BlockSpec((BLK, H), lambda i, li=li: (i, li)),
            pl.BlockSpec((G, H), lambda i: (0, 0)),
        ],
        out_shape=[
            jax.ShapeDtypeStruct((N, L * H), jnp.float32),
            jax.ShapeDtypeStruct((G, H), jnp.float32),
        ],
        input_output_aliases=aliases,
    )(*inputs)


def kernel(x, edge_index, batch,
           W1_0, b1_0, W2_0, b2_0, gamma_0, beta_0,
           W1_1, b1_1, W2_1, b2_1, gamma_1, beta_1,
           W1_2, b1_2, W2_2, b2_2, gamma_2, beta_2):
    src = edge_index[0].reshape(NW, EDGES_PER_TILE)
    dst = edge_index[1].reshape(NW, EDGES_PER_TILE)
    batch3 = batch.reshape(NBLK, 1, BLK)

    agg0 = _sc_agg(x, src, dst)
    m0, s0 = _tc_mlp0(x, agg0, W1_0, b1_0, W2_0, b2_0)
    indeg = _sc_agg(jnp.ones((N, H), jnp.float32), src, dst)

    agg1 = _sc_agg(m0, src, dst)
    xs, pool0 = _tc_norm_pool(m0, s0, gamma_0, beta_0, batch3, None, 0)
    m1, s1 = _tc_mlpN(m0, agg1, s0, indeg, gamma_0, beta_0,
                      W1_1, b1_1, W2_1, b2_1)

    agg2 = _sc_agg(m1, src, dst)
    xs, pool1 = _tc_norm_pool(m1, s1, gamma_1, beta_1, batch3, xs, 1)
    m2, s2 = _tc_mlpN(m1, agg2, s1, indeg, gamma_1, beta_1,
                      W1_2, b1_2, W2_2, b2_2)

    xs, pool2 = _tc_norm_pool(m2, s2, gamma_2, beta_2, batch3, xs, 2)

    return (jnp.concatenate([pool0, pool1, pool2], axis=1), xs)
